# baseline (speedup 1.0000x reference)
import jax
import jax.numpy as jnp
from jax.experimental import pallas as pl
from jax.experimental.pallas import tpu as pltpu

_N, _D = 16384, 64
_NC = 32
_CH = _N // _NC


def _body(ind_hbm, x_hbm, w_hbm, b_hbm, o_hbm,
          ind_v, w_v, b_v, xbuf, obuf,
          insem, outsem, csem):
    pltpu.make_async_copy(w_hbm, w_v, csem.at[0]).start()
    pltpu.make_async_copy(b_hbm, b_v, csem.at[1]).start()
    pltpu.make_async_copy(ind_hbm, ind_v, csem.at[2]).start()
    for c in range(_NC):
        pltpu.make_async_copy(
            x_hbm.at[pl.ds(c * _CH, _CH)], xbuf.at[c], insem.at[c]
        ).start()
    pltpu.make_async_copy(w_hbm, w_v, csem.at[0]).wait()
    pltpu.make_async_copy(b_hbm, b_v, csem.at[1]).wait()
    pltpu.make_async_copy(ind_hbm, ind_v, csem.at[2]).wait()
    w = w_v[...]
    bias = b_v[...]
    for c in range(_NC):
        pltpu.make_async_copy(
            x_hbm.at[pl.ds(c * _CH, _CH)], xbuf.at[c], insem.at[c]
        ).wait()
        acc = jnp.dot(xbuf[c], w, preferred_element_type=jnp.float32) + bias
        mrow = jnp.where(ind_v[:, pl.ds(c * _CH, _CH)] != 0, 1.0, 0.0)
        obuf[c] = acc * jnp.transpose(mrow)
        pltpu.make_async_copy(
            obuf.at[c], o_hbm.at[pl.ds(c * _CH, _CH)], outsem.at[c]
        ).start()
    for c in range(_NC):
        pltpu.make_async_copy(
            obuf.at[c], o_hbm.at[pl.ds(c * _CH, _CH)], outsem.at[c]
        ).wait()


def kernel(indicators, data, W, b):
    N, D = data.shape
    return pl.pallas_call(
        _body,
        in_specs=[
            pl.BlockSpec(memory_space=pl.ANY),
            pl.BlockSpec(memory_space=pl.ANY),
            pl.BlockSpec(memory_space=pl.ANY),
            pl.BlockSpec(memory_space=pl.ANY),
        ],
        out_specs=pl.BlockSpec(memory_space=pl.ANY),
        out_shape=jax.ShapeDtypeStruct((N, D), jnp.float32),
        scratch_shapes=[
            pltpu.VMEM((1, _N), jnp.int32),
            pltpu.VMEM((D, D), jnp.float32),
            pltpu.VMEM((1, D), jnp.float32),
            pltpu.VMEM((_NC, _CH, _D), jnp.float32),
            pltpu.VMEM((_NC, _CH, _D), jnp.float32),
            pltpu.SemaphoreType.DMA((_NC,)),
            pltpu.SemaphoreType.DMA((_NC,)),
            pltpu.SemaphoreType.DMA((3,)),
        ],
    )(indicators.reshape(1, N), data, W, b.reshape(1, D))


# speedup vs baseline: 1.1645x; 1.1645x over previous
import jax
import jax.numpy as jnp
from jax.experimental import pallas as pl
from jax.experimental.pallas import tpu as pltpu

_N, _D = 16384, 64
_NC = 8
_CH = _N // _NC
_NP = 4
_SP = _CH // _NP


def _body(ind_hbm, x_hbm, w_hbm, b_hbm, o_hbm,
          ind_v, w_v, b_v, xbuf, obuf,
          insem, outsem, csem):
    pltpu.make_async_copy(w_hbm, w_v, csem.at[0]).start()
    pltpu.make_async_copy(b_hbm, b_v, csem.at[1]).start()
    pltpu.make_async_copy(ind_hbm, ind_v, csem.at[2]).start()
    for c in range(_NC):
        pltpu.make_async_copy(
            x_hbm.at[pl.ds(c * _CH, _CH)], xbuf.at[c], insem.at[c]
        ).start()
    pltpu.make_async_copy(w_hbm, w_v, csem.at[0]).wait()
    pltpu.make_async_copy(b_hbm, b_v, csem.at[1]).wait()
    pltpu.make_async_copy(ind_hbm, ind_v, csem.at[2]).wait()
    w = w_v[...]
    bias = b_v[...]
    for c in range(_NC):
        pltpu.make_async_copy(
            x_hbm.at[pl.ds(c * _CH, _CH)], xbuf.at[c], insem.at[c]
        ).wait()
        for p in range(_NP):
            x = xbuf[c, pl.ds(p * _SP, _SP), :]
            acc = jnp.dot(x, w, preferred_element_type=jnp.float32) + bias
            mrow = jnp.where(
                ind_v[:, pl.ds(c * _CH + p * _SP, _SP)] != 0, 1.0, 0.0
            )
            obuf[c, pl.ds(p * _SP, _SP), :] = acc * jnp.transpose(mrow)
            pltpu.make_async_copy(
                obuf.at[c, pl.ds(p * _SP, _SP)],
                o_hbm.at[pl.ds(c * _CH + p * _SP, _SP)],
                outsem.at[c * _NP + p],
            ).start()
    for c in range(_NC):
        for p in range(_NP):
            pltpu.make_async_copy(
                obuf.at[c, pl.ds(p * _SP, _SP)],
                o_hbm.at[pl.ds(c * _CH + p * _SP, _SP)],
                outsem.at[c * _NP + p],
            ).wait()


def kernel(indicators, data, W, b):
    N, D = data.shape
    return pl.pallas_call(
        _body,
        in_specs=[
            pl.BlockSpec(memory_space=pl.ANY),
            pl.BlockSpec(memory_space=pl.ANY),
            pl.BlockSpec(memory_space=pl.ANY),
            pl.BlockSpec(memory_space=pl.ANY),
        ],
        out_specs=pl.BlockSpec(memory_space=pl.ANY),
        out_shape=jax.ShapeDtypeStruct((N, D), jnp.float32),
        scratch_shapes=[
            pltpu.VMEM((1, _N), jnp.int32),
            pltpu.VMEM((D, D), jnp.float32),
            pltpu.VMEM((1, D), jnp.float32),
            pltpu.VMEM((_NC, _CH, _D), jnp.float32),
            pltpu.VMEM((_NC, _CH, _D), jnp.float32),
            pltpu.SemaphoreType.DMA((_NC,)),
            pltpu.SemaphoreType.DMA((_NC * _NP,)),
            pltpu.SemaphoreType.DMA((3,)),
        ],
    )(indicators.reshape(1, N), data, W, b.reshape(1, D))


# speedup vs baseline: 1.1651x; 1.0006x over previous
import jax
import jax.numpy as jnp
from jax.experimental import pallas as pl
from jax.experimental.pallas import tpu as pltpu

_N, _D = 16384, 64
_NC = 8
_CH = _N // _NC
_NP = 4
_SP = _CH // _NP


def _body(ind_hbm, x_hbm, w_hbm, b_hbm, o_hbm,
          ind_v, w_v, b_v, xbuf, obuf,
          insem, outsem, csem):
    pltpu.make_async_copy(w_hbm, w_v, csem.at[0]).start()
    pltpu.make_async_copy(b_hbm, b_v, csem.at[1]).start()
    pltpu.make_async_copy(ind_hbm, ind_v, csem.at[2]).start()
    for c in range(_NC):
        pltpu.make_async_copy(
            x_hbm.at[pl.ds(c * _CH, _CH)], xbuf.at[c], insem.at[c]
        ).start()
    pltpu.make_async_copy(w_hbm, w_v, csem.at[0]).wait()
    pltpu.make_async_copy(b_hbm, b_v, csem.at[1]).wait()
    pltpu.make_async_copy(ind_hbm, ind_v, csem.at[2]).wait()
    wb = w_v[...].astype(jnp.bfloat16)
    bias = b_v[...]
    for c in range(_NC):
        pltpu.make_async_copy(
            x_hbm.at[pl.ds(c * _CH, _CH)], xbuf.at[c], insem.at[c]
        ).wait()
        for p in range(_NP):
            x = xbuf[c, pl.ds(p * _SP, _SP), :].astype(jnp.bfloat16)
            acc = jnp.dot(x, wb, preferred_element_type=jnp.float32) + bias
            mrow = jnp.where(
                ind_v[:, pl.ds(c * _CH + p * _SP, _SP)] != 0, 1.0, 0.0
            )
            obuf[c, pl.ds(p * _SP, _SP), :] = acc * jnp.transpose(mrow)
            pltpu.make_async_copy(
                obuf.at[c, pl.ds(p * _SP, _SP)],
                o_hbm.at[pl.ds(c * _CH + p * _SP, _SP)],
                outsem.at[c * _NP + p],
            ).start()
    for c in range(_NC):
        for p in range(_NP):
            pltpu.make_async_copy(
                obuf.at[c, pl.ds(p * _SP, _SP)],
                o_hbm.at[pl.ds(c * _CH + p * _SP, _SP)],
                outsem.at[c * _NP + p],
            ).wait()


def kernel(indicators, data, W, b):
    N, D = data.shape
    return pl.pallas_call(
        _body,
        in_specs=[
            pl.BlockSpec(memory_space=pl.ANY),
            pl.BlockSpec(memory_space=pl.ANY),
            pl.BlockSpec(memory_space=pl.ANY),
            pl.BlockSpec(memory_space=pl.ANY),
        ],
        out_specs=pl.BlockSpec(memory_space=pl.ANY),
        out_shape=jax.ShapeDtypeStruct((N, D), jnp.float32),
        scratch_shapes=[
            pltpu.VMEM((1, _N), jnp.int32),
            pltpu.VMEM((D, D), jnp.float32),
            pltpu.VMEM((1, D), jnp.float32),
            pltpu.VMEM((_NC, _CH, _D), jnp.float32),
            pltpu.VMEM((_NC, _CH, _D), jnp.float32),
            pltpu.SemaphoreType.DMA((_NC,)),
            pltpu.SemaphoreType.DMA((_NC * _NP,)),
            pltpu.SemaphoreType.DMA((3,)),
        ],
    )(indicators.reshape(1, N), data, W, b.reshape(1, D))
